# Initial kernel scaffold; baseline (speedup 1.0000x reference)
import jax
import jax.numpy as jnp
from jax.experimental import pallas as pl


def kernel(x, enc_w1, enc_b1, enc_w2, enc_b2, lat_w, lat_b, codebook, dec_w1, dec_b1, dec_w2, dec_b2, out_w, out_b):
    raise NotImplementedError("write your pallas kernel here")


# speedup vs baseline: 5.0169x; 5.0169x over previous
import functools

import jax
import jax.numpy as jnp
from jax.experimental import pallas as pl

_INV_SQRT2 = 0.7071067811865476


def _gelu(v):
    return 0.5 * v * (1.0 + jax.lax.erf(v * _INV_SQRT2))


def _body(x_ref, w1_ref, b1_ref, w2_ref, b2_ref, lw_ref, lb_ref,
          cb_ref, cbt_ref, dw1_ref, db1_ref, dw2_ref, db2_ref,
          ow_ref, ob_ref, zq_ref, part_ref):
    x = x_ref[...]
    h = _gelu(jnp.dot(x, w1_ref[...], preferred_element_type=jnp.float32)
              + b1_ref[...])
    h = _gelu(jnp.dot(h, w2_ref[...], preferred_element_type=jnp.float32)
              + b2_ref[...])
    z_e = jnp.dot(h, lw_ref[...], preferred_element_type=jnp.float32) \
        + lb_ref[...]

    cbt = cbt_ref[...]
    cb_norm = jnp.sum(cbt * cbt, axis=0, keepdims=True)
    scores = jnp.dot(z_e, cbt, preferred_element_type=jnp.float32,
                     precision=jax.lax.Precision.HIGHEST)
    d2 = cb_norm - 2.0 * scores
    K = d2.shape[1]
    min_val = jnp.min(d2, axis=1, keepdims=True)
    k_iota = jax.lax.broadcasted_iota(jnp.int32, d2.shape, 1)
    masked = jnp.where(d2 <= min_val, k_iota, K)
    idx = jnp.min(masked, axis=1, keepdims=True)
    one_hot = (k_iota == idx).astype(jnp.float32)
    z_q = jnp.dot(one_hot, cb_ref[...], preferred_element_type=jnp.float32,
                  precision=jax.lax.Precision.HIGHEST)
    zq_ref[...] = z_q

    r = _gelu(jnp.dot(z_q, dw1_ref[...], preferred_element_type=jnp.float32)
              + db1_ref[...])
    r = _gelu(jnp.dot(r, dw2_ref[...], preferred_element_type=jnp.float32)
              + db2_ref[...])
    x_recon = jnp.dot(r, ow_ref[...], preferred_element_type=jnp.float32) \
        + ob_ref[...]

    dr = x_recon - x
    dq = z_q - z_e
    recon_part = jnp.sum(dr * dr, axis=0, keepdims=True)
    q_part = jnp.sum(dq * dq, axis=0, keepdims=True)

    @pl.when(pl.program_id(0) == 0)
    def _init():
        part_ref[...] = jnp.zeros_like(part_ref)

    part_ref[0:1, 0:recon_part.shape[1]] += recon_part
    part_ref[1:2, 0:q_part.shape[1]] += q_part


@functools.partial(jax.jit, static_argnames=())
def kernel(x, enc_w1, enc_b1, enc_w2, enc_b2, lat_w, lat_b, codebook,
           dec_w1, dec_b1, dec_w2, dec_b2, out_w, out_b):
    B, S, F = x.shape
    N = B * S
    L = codebook.shape[1]
    T = 576
    grid = (N // T,)

    x_flat = x.reshape(N, F)
    w1 = enc_w1.T
    w2 = enc_w2.T
    lw = lat_w.T
    cbt = codebook.T
    dw1 = dec_w1.T
    dw2 = dec_w2.T
    ow = out_w.T
    b1 = enc_b1[None, :]
    b2 = enc_b2[None, :]
    lb = lat_b[None, :]
    db1 = dec_b1[None, :]
    db2 = dec_b2[None, :]
    ob = out_b[None, :]

    def full(a):
        return pl.BlockSpec(a.shape, lambda i: (0, 0))

    z_q, part = pl.pallas_call(
        _body,
        grid=grid,
        in_specs=[
            pl.BlockSpec((T, F), lambda i: (i, 0)),
            full(w1), full(b1), full(w2), full(b2), full(lw), full(lb),
            full(codebook), full(cbt), full(dw1), full(db1), full(dw2),
            full(db2), full(ow), full(ob),
        ],
        out_specs=[
            pl.BlockSpec((T, L), lambda i: (i, 0)),
            pl.BlockSpec((2, F), lambda i: (0, 0)),
        ],
        out_shape=[
            jax.ShapeDtypeStruct((N, L), jnp.float32),
            jax.ShapeDtypeStruct((2, F), jnp.float32),
        ],
    )(x_flat, w1, b1, w2, b2, lw, lb, codebook, cbt, dw1, db1, dw2, db2,
      ow, ob)

    z_latent = z_q.reshape(B, S, L)
    loss = jnp.sum(part[0]) / (N * F) + 0.5 * (jnp.sum(part[1]) / (N * L))
    return (z_latent, loss)


# speedup vs baseline: 6.9069x; 1.3767x over previous
import functools

import jax
import jax.numpy as jnp
from jax.experimental import pallas as pl

_INV_SQRT2 = 0.7071067811865476


def _gelu(v):
    return 0.5 * v * (1.0 + jax.lax.erf(v * _INV_SQRT2))


def _body(x_ref, w1_ref, b1_ref, w2_ref, b2_ref, lw_ref, lb_ref,
          cbh_ref, cbm_ref, cbl_ref, cbt_ref, dw1_ref, db1_ref, dw2_ref,
          db2_ref, ow_ref, ob_ref, zq_ref, part_ref):
    x = x_ref[...]
    h = _gelu(jnp.dot(x, w1_ref[...], preferred_element_type=jnp.float32)
              + b1_ref[...])
    h = _gelu(jnp.dot(h, w2_ref[...], preferred_element_type=jnp.float32)
              + b2_ref[...])
    z_e = jnp.dot(h, lw_ref[...], preferred_element_type=jnp.float32) \
        + lb_ref[...]

    cbt = cbt_ref[...]
    cb_norm = jnp.sum(cbt * cbt, axis=0, keepdims=True)
    scores = jnp.dot(z_e, cbt, preferred_element_type=jnp.float32,
                     precision=jax.lax.Precision.HIGHEST)
    d2 = cb_norm - 2.0 * scores
    K = d2.shape[1]
    min_val = jnp.min(d2, axis=1, keepdims=True)
    k_iota = jax.lax.broadcasted_iota(jnp.int32, d2.shape, 1)
    masked = jnp.where(d2 <= min_val, k_iota, K)
    idx = jnp.min(masked, axis=1, keepdims=True)
    one_hot = (k_iota == idx).astype(jnp.bfloat16)
    z_q = (jnp.dot(one_hot, cbh_ref[...], preferred_element_type=jnp.float32)
           + jnp.dot(one_hot, cbm_ref[...], preferred_element_type=jnp.float32)
           + jnp.dot(one_hot, cbl_ref[...], preferred_element_type=jnp.float32))
    zq_ref[...] = z_q

    r = _gelu(jnp.dot(z_q, dw1_ref[...], preferred_element_type=jnp.float32)
              + db1_ref[...])
    r = _gelu(jnp.dot(r, dw2_ref[...], preferred_element_type=jnp.float32)
              + db2_ref[...])
    x_recon = jnp.dot(r, ow_ref[...], preferred_element_type=jnp.float32) \
        + ob_ref[...]

    dr = x_recon - x
    dq = z_q - z_e
    recon_part = jnp.sum(dr * dr, axis=0, keepdims=True)
    q_part = jnp.sum(dq * dq, axis=0, keepdims=True)

    @pl.when(pl.program_id(0) == 0)
    def _init():
        part_ref[...] = jnp.zeros_like(part_ref)

    part_ref[0:1, 0:recon_part.shape[1]] += recon_part
    part_ref[1:2, 0:q_part.shape[1]] += q_part


@functools.partial(jax.jit, static_argnames=())
def kernel(x, enc_w1, enc_b1, enc_w2, enc_b2, lat_w, lat_b, codebook,
           dec_w1, dec_b1, dec_w2, dec_b2, out_w, out_b):
    B, S, F = x.shape
    N = B * S
    L = codebook.shape[1]
    T = 576
    grid = (N // T,)

    x_flat = x.reshape(N, F)
    w1 = enc_w1.T
    w2 = enc_w2.T
    lw = lat_w.T
    cbt = codebook.T
    dw1 = dec_w1.T
    dw2 = dec_w2.T
    ow = out_w.T
    cbh = codebook.astype(jnp.bfloat16)
    cbm = (codebook - cbh.astype(jnp.float32)).astype(jnp.bfloat16)
    cbl = ((codebook - cbh.astype(jnp.float32))
           - cbm.astype(jnp.float32)).astype(jnp.bfloat16)
    b1 = enc_b1[None, :]
    b2 = enc_b2[None, :]
    lb = lat_b[None, :]
    db1 = dec_b1[None, :]
    db2 = dec_b2[None, :]
    ob = out_b[None, :]

    def full(a):
        return pl.BlockSpec(a.shape, lambda i: (0, 0))

    z_q, part = pl.pallas_call(
        _body,
        grid=grid,
        in_specs=[
            pl.BlockSpec((T, F), lambda i: (i, 0)),
            full(w1), full(b1), full(w2), full(b2), full(lw), full(lb),
            full(cbh), full(cbm), full(cbl), full(cbt), full(dw1),
            full(db1), full(dw2), full(db2), full(ow), full(ob),
        ],
        out_specs=[
            pl.BlockSpec((T, L), lambda i: (i, 0)),
            pl.BlockSpec((2, F), lambda i: (0, 0)),
        ],
        out_shape=[
            jax.ShapeDtypeStruct((N, L), jnp.float32),
            jax.ShapeDtypeStruct((2, F), jnp.float32),
        ],
    )(x_flat, w1, b1, w2, b2, lw, lb, cbh, cbm, cbl, cbt, dw1, db1, dw2,
      db2, ow, ob)

    z_latent = z_q.reshape(B, S, L)
    loss = jnp.sum(part[0]) / (N * F) + 0.5 * (jnp.sum(part[1]) / (N * L))
    return (z_latent, loss)


# speedup vs baseline: 7.3240x; 1.0604x over previous
import functools

import jax
import jax.numpy as jnp
from jax.experimental import pallas as pl

_INV_SQRT2 = 0.7071067811865476


def _gelu(v):
    return 0.5 * v * (1.0 + jax.lax.erf(v * _INV_SQRT2))


def _body(x_ref, w1_ref, b1_ref, w2_ref, b2_ref, lw_ref, lb_ref,
          cbh_ref, cbm_ref, cbl_ref, cbt_ref,
          dw1_ref, db1_ref, dw2_ref, db2_ref, ow_ref, ob_ref,
          zq_ref, part_ref):
    x = x_ref[...]
    h = _gelu(jnp.dot(x, w1_ref[...], preferred_element_type=jnp.float32)
              + b1_ref[...])
    h = _gelu(jnp.dot(h, w2_ref[...], preferred_element_type=jnp.float32)
              + b2_ref[...])
    z_e = jnp.dot(h, lw_ref[...], preferred_element_type=jnp.float32) \
        + lb_ref[...]

    scores = jnp.dot(z_e, cbt_ref[...], preferred_element_type=jnp.float32,
                     precision=jax.lax.Precision.HIGHEST)
    cbt = cbt_ref[...]
    cb_norm = jnp.sum(cbt * cbt, axis=0, keepdims=True)
    d2 = cb_norm - 2.0 * scores
    K = d2.shape[1]
    min_val = jnp.min(d2, axis=1, keepdims=True)
    k_iota = jax.lax.broadcasted_iota(jnp.int32, d2.shape, 1)
    masked = jnp.where(d2 <= min_val, k_iota, K)
    idx = jnp.min(masked, axis=1, keepdims=True)
    one_hot = (k_iota == idx).astype(jnp.bfloat16)
    z_q = (jnp.dot(one_hot, cbh_ref[...], preferred_element_type=jnp.float32)
           + jnp.dot(one_hot, cbm_ref[...], preferred_element_type=jnp.float32)
           + jnp.dot(one_hot, cbl_ref[...], preferred_element_type=jnp.float32))
    zq_ref[...] = z_q

    r = _gelu(jnp.dot(z_q, dw1_ref[...], preferred_element_type=jnp.float32)
              + db1_ref[...])
    r = _gelu(jnp.dot(r, dw2_ref[...], preferred_element_type=jnp.float32)
              + db2_ref[...])
    x_recon = jnp.dot(r, ow_ref[...], preferred_element_type=jnp.float32) \
        + ob_ref[...]

    dr = x_recon - x
    dq = z_q - z_e
    recon_part = jnp.sum(dr * dr, axis=0, keepdims=True)
    q_part = jnp.sum(dq * dq, axis=0, keepdims=True)

    @pl.when(pl.program_id(0) == 0)
    def _init():
        part_ref[...] = jnp.zeros_like(part_ref)

    part_ref[0:1, 0:recon_part.shape[1]] += recon_part
    part_ref[1:2, 0:q_part.shape[1]] += q_part


@functools.partial(jax.jit, static_argnames=())
def kernel(x, enc_w1, enc_b1, enc_w2, enc_b2, lat_w, lat_b, codebook,
           dec_w1, dec_b1, dec_w2, dec_b2, out_w, out_b):
    B, S, F = x.shape
    N = B * S
    L = codebook.shape[1]
    T = 1152
    grid = (N // T,)

    x_flat = x.reshape(N, F)
    w1 = enc_w1.T
    w2 = enc_w2.T
    lw = lat_w.T
    cbt = codebook.T
    dw1 = dec_w1.T
    dw2 = dec_w2.T
    ow = out_w.T
    cbh = codebook.astype(jnp.bfloat16)
    cbm = (codebook - cbh.astype(jnp.float32)).astype(jnp.bfloat16)
    cbl = ((codebook - cbh.astype(jnp.float32))
           - cbm.astype(jnp.float32)).astype(jnp.bfloat16)
    b1 = enc_b1[None, :]
    b2 = enc_b2[None, :]
    lb = lat_b[None, :]
    db1 = dec_b1[None, :]
    db2 = dec_b2[None, :]
    ob = out_b[None, :]

    def full(a):
        return pl.BlockSpec(a.shape, lambda i: (0, 0))

    z_q, part = pl.pallas_call(
        _body,
        grid=grid,
        in_specs=[
            pl.BlockSpec((T, F), lambda i: (i, 0)),
            full(w1), full(b1), full(w2), full(b2), full(lw), full(lb),
            full(cbh), full(cbm), full(cbl), full(cbt), full(dw1),
            full(db1), full(dw2), full(db2), full(ow), full(ob),
        ],
        out_specs=[
            pl.BlockSpec((T, L), lambda i: (i, 0)),
            pl.BlockSpec((2, F), lambda i: (0, 0)),
        ],
        out_shape=[
            jax.ShapeDtypeStruct((N, L), jnp.float32),
            jax.ShapeDtypeStruct((2, F), jnp.float32),
        ],
    )(x_flat, w1, b1, w2, b2, lw, lb, cbh, cbm, cbl, cbt,
      dw1, db1, dw2, db2, ow, ob)

    z_latent = z_q.reshape(B, S, L)
    loss = jnp.sum(part[0]) / (N * F) + 0.5 * (jnp.sum(part[1]) / (N * L))
    return (z_latent, loss)


# speedup vs baseline: 7.3606x; 1.0050x over previous
import functools

import jax
import jax.numpy as jnp
from jax.experimental import pallas as pl

_INV_SQRT2 = 0.7071067811865476


def _gelu(v):
    return 0.5 * v * (1.0 + jax.lax.erf(v * _INV_SQRT2))


def _body(x_ref, w1_ref, b1_ref, w2_ref, b2_ref, lw_ref, lb_ref,
          cbh_ref, cbm_ref, cbl_ref, cbt_ref,
          dw1_ref, db1_ref, dw2_ref, db2_ref, ow_ref, ob_ref,
          zq_ref, part_ref):
    x = x_ref[...]
    h = _gelu(jnp.dot(x, w1_ref[...], preferred_element_type=jnp.float32)
              + b1_ref[...])
    h = _gelu(jnp.dot(h, w2_ref[...], preferred_element_type=jnp.float32)
              + b2_ref[...])
    z_e = jnp.dot(h, lw_ref[...], preferred_element_type=jnp.float32) \
        + lb_ref[...]

    scores = jnp.dot(z_e, cbt_ref[...], preferred_element_type=jnp.float32,
                     precision=jax.lax.Precision.HIGHEST)
    cbt = cbt_ref[...]
    cb_norm = jnp.sum(cbt * cbt, axis=0, keepdims=True)
    d2 = cb_norm - 2.0 * scores
    K = d2.shape[1]
    min_val = jnp.min(d2, axis=1, keepdims=True)
    k_iota = jax.lax.broadcasted_iota(jnp.int32, d2.shape, 1)
    masked = jnp.where(d2 <= min_val, k_iota, K)
    idx = jnp.min(masked, axis=1, keepdims=True)
    one_hot = (k_iota == idx).astype(jnp.bfloat16)
    z_q = (jnp.dot(one_hot, cbh_ref[...], preferred_element_type=jnp.float32)
           + jnp.dot(one_hot, cbm_ref[...], preferred_element_type=jnp.float32)
           + jnp.dot(one_hot, cbl_ref[...], preferred_element_type=jnp.float32))
    zq_ref[...] = z_q

    r = _gelu(jnp.dot(z_q, dw1_ref[...], preferred_element_type=jnp.float32)
              + db1_ref[...])
    r = _gelu(jnp.dot(r, dw2_ref[...], preferred_element_type=jnp.float32)
              + db2_ref[...])
    x_recon = jnp.dot(r, ow_ref[...], preferred_element_type=jnp.float32) \
        + ob_ref[...]

    dr = x_recon - x
    dq = z_q - z_e
    recon_part = jnp.sum(dr * dr, axis=0, keepdims=True)
    q_part = jnp.sum(dq * dq, axis=0, keepdims=True)

    @pl.when(pl.program_id(0) == 0)
    def _init():
        part_ref[...] = jnp.zeros_like(part_ref)

    part_ref[0:1, 0:recon_part.shape[1]] += recon_part
    part_ref[1:2, 0:q_part.shape[1]] += q_part


@functools.partial(jax.jit, static_argnames=())
def kernel(x, enc_w1, enc_b1, enc_w2, enc_b2, lat_w, lat_b, codebook,
           dec_w1, dec_b1, dec_w2, dec_b2, out_w, out_b):
    B, S, F = x.shape
    N = B * S
    L = codebook.shape[1]
    T = 2304
    grid = (N // T,)

    x_flat = x.reshape(N, F)
    w1 = enc_w1.T
    w2 = enc_w2.T
    lw = lat_w.T
    cbt = codebook.T
    dw1 = dec_w1.T
    dw2 = dec_w2.T
    ow = out_w.T
    cbh = codebook.astype(jnp.bfloat16)
    cbm = (codebook - cbh.astype(jnp.float32)).astype(jnp.bfloat16)
    cbl = ((codebook - cbh.astype(jnp.float32))
           - cbm.astype(jnp.float32)).astype(jnp.bfloat16)
    b1 = enc_b1[None, :]
    b2 = enc_b2[None, :]
    lb = lat_b[None, :]
    db1 = dec_b1[None, :]
    db2 = dec_b2[None, :]
    ob = out_b[None, :]

    def full(a):
        return pl.BlockSpec(a.shape, lambda i: (0, 0))

    z_q, part = pl.pallas_call(
        _body,
        grid=grid,
        in_specs=[
            pl.BlockSpec((T, F), lambda i: (i, 0)),
            full(w1), full(b1), full(w2), full(b2), full(lw), full(lb),
            full(cbh), full(cbm), full(cbl), full(cbt), full(dw1),
            full(db1), full(dw2), full(db2), full(ow), full(ob),
        ],
        out_specs=[
            pl.BlockSpec((T, L), lambda i: (i, 0)),
            pl.BlockSpec((2, F), lambda i: (0, 0)),
        ],
        out_shape=[
            jax.ShapeDtypeStruct((N, L), jnp.float32),
            jax.ShapeDtypeStruct((2, F), jnp.float32),
        ],
    )(x_flat, w1, b1, w2, b2, lw, lb, cbh, cbm, cbl, cbt,
      dw1, db1, dw2, db2, ow, ob)

    z_latent = z_q.reshape(B, S, L)
    loss = jnp.sum(part[0]) / (N * F) + 0.5 * (jnp.sum(part[1]) / (N * L))
    return (z_latent, loss)
